# baseline (speedup 1.0000x reference)
import functools

import jax
import jax.numpy as jnp
from jax.experimental import pallas as pl
from jax.experimental.pallas import tpu as pltpu
from jax.experimental.pallas import tpu_sc as plsc

N = 4096
D_IN = 2
H = 256
Z = 128
DMAX = 64
NEG = -1e9
BIG = 1e9

BV = 256
BU = 256
GCH = 64


def _fused_body(x_ref, z_ref, ew1_ref, eb1_ref, ew2_ref, eb2_ref, pw_ref,
                pb_ref, sw_ref, tw_ref, dcol_ref, dvm_ref, dsm_ref,
                h_out, vals_ref, idx_ref, idxf_ref,
                h_s, src_s, tgt_s, srcd_s):
    i = pl.program_id(0)
    dn = (((1,), (1,)), ((), ()))

    @pl.when(i == 0)
    def _dense():
        x = x_ref[...]
        h0 = jnp.maximum(
            jax.lax.dot_general(x, ew1_ref[...], dn,
                                preferred_element_type=jnp.float32)
            + eb1_ref[...], 0.0)
        h0 = jax.lax.dot_general(h0, ew2_ref[...], dn,
                                 preferred_element_type=jnp.float32) \
            + eb2_ref[...]
        zrow = jnp.broadcast_to(z_ref[...], (N, Z))
        hcat = jnp.concatenate([h0, zrow], axis=1)
        h = jnp.maximum(
            jax.lax.dot_general(hcat, pw_ref[...], dn,
                                preferred_element_type=jnp.float32)
            + pb_ref[...], 0.0)
        h_s[...] = h
        h_out[...] = h
        src_s[...] = jax.lax.dot_general(h, sw_ref[...], dn,
                                         preferred_element_type=jnp.float32)
        tgt_s[...] = jax.lax.dot_general(h, tw_ref[...], dn,
                                         preferred_element_type=jnp.float32)
        du = dcol_ref[...]
        jio = jax.lax.broadcasted_iota(jnp.int32, (N, DMAX), 1)
        srcd_s[...] = jnp.where((du <= jio) & (jio <= du + 3), 0.0, -1.0)

    v0 = i * BV
    d_all = dvm_ref[...]
    d0 = dsm_ref[0, v0]
    d1 = dsm_ref[0, v0 + BV - 1]
    lo_u = jnp.sum((d_all < d0 - 3).astype(jnp.int32))
    hi_u = N - jnp.sum((d_all > d1).astype(jnp.int32))
    ublo = lo_u // BU
    ubhi = (hi_u + BU - 1) // BU

    tgt = tgt_s[pl.ds(v0, BV), :]
    dvb = dcol_ref[pl.ds(v0, BV), :]
    tgtd = jnp.where(
        jax.lax.broadcasted_iota(jnp.int32, (BV, DMAX), 1) == dvb, BIG, 0.0)
    rowio = v0 + jax.lax.broadcasted_iota(jnp.int32, (BV, 1), 0)

    def step(ub, carry):
        b1v, b1i, b2v, b2i = carry
        u0 = ub * BU
        srcb = src_s[pl.ds(u0, BU), :]
        srcdb = srcd_s[pl.ds(u0, BU), :]
        s = jax.lax.dot_general(tgt, srcb, dn,
                                preferred_element_type=jnp.float32) \
            + jax.lax.dot_general(tgtd, srcdb, dn,
                                  preferred_element_type=jnp.float32)
        uidx = u0 + jax.lax.broadcasted_iota(jnp.int32, (BV, BU), 1)
        selfcol = jnp.where(u0 == v0, rowio, -1)
        sm = jnp.where(uidx == selfcol, NEG, s)
        t1v = jnp.max(sm, axis=1, keepdims=True)
        t1i = jnp.min(jnp.where(sm == t1v, uidx, N), axis=1, keepdims=True)
        sm2 = jnp.where(uidx == t1i, -jnp.inf, sm)
        t2v = jnp.max(sm2, axis=1, keepdims=True)
        t2i = jnp.min(jnp.where(sm2 == t2v, uidx, N), axis=1, keepdims=True)
        take_b = b1v >= t1v
        n1v = jnp.where(take_b, b1v, t1v)
        n1i = jnp.where(take_b, b1i, t1i)
        av = jnp.where(take_b, b2v, b1v)
        ai = jnp.where(take_b, b2i, b1i)
        cv = jnp.where(take_b, t1v, t2v)
        ci = jnp.where(take_b, t1i, t2i)
        take_a = av >= cv
        n2v = jnp.where(take_a, av, cv)
        n2i = jnp.where(take_a, ai, ci)
        return n1v, n1i, n2v, n2i

    init = (jnp.full((BV, 1), NEG, jnp.float32),
            jnp.zeros((BV, 1), jnp.int32),
            jnp.full((BV, 1), NEG, jnp.float32),
            jnp.zeros((BV, 1), jnp.int32))
    b1v, b1i, b2v, b2i = jax.lax.fori_loop(ublo, ubhi, step, init)

    no1 = b1v <= -1e8
    b1i = jnp.where(no1, 0, b1i)
    no2 = (~no1) & (b2v <= -1e8)
    b2i = jnp.where(no1, 1,
                    jnp.where(no2, jnp.where(b1i == 0, 1, 0), b2i))
    b1v = jnp.where(no1, NEG, b1v)
    b2v = jnp.where(no1 | no2, NEG, b2v)

    vals_ref[:, 0:1] = b1v
    vals_ref[:, 1:2] = b2v
    idx_ref[:, 0:1] = b1i
    idx_ref[:, 1:2] = b2i
    idxf_ref[0:1, :] = jnp.transpose(b1i)
    idxf_ref[1:2, :] = jnp.transpose(b2i)


def _fused_call(x, z2, ew1, eb1, ew2, eb2, pw, pb, sw, tw, dcol, d2):
    cst = lambda i: (0, 0)
    return pl.pallas_call(
        _fused_body,
        grid=(N // BV,),
        in_specs=[
            pl.BlockSpec((N, D_IN), cst),
            pl.BlockSpec((1, Z), cst),
            pl.BlockSpec((H, D_IN), cst),
            pl.BlockSpec((1, H), cst),
            pl.BlockSpec((H, H), cst),
            pl.BlockSpec((1, H), cst),
            pl.BlockSpec((H, H + Z), cst),
            pl.BlockSpec((1, H), cst),
            pl.BlockSpec((H, H), cst),
            pl.BlockSpec((H, H), cst),
            pl.BlockSpec((N, 1), cst),
            pl.BlockSpec((1, N), cst),
            pl.BlockSpec(memory_space=pltpu.SMEM),
        ],
        out_specs=[
            pl.BlockSpec((N, H), cst),
            pl.BlockSpec((BV, 2), lambda i: (i, 0)),
            pl.BlockSpec((BV, 2), lambda i: (i, 0)),
            pl.BlockSpec((2, BV), lambda i: (0, i)),
        ],
        out_shape=(jax.ShapeDtypeStruct((N, H), jnp.float32),
                   jax.ShapeDtypeStruct((N, 2), jnp.float32),
                   jax.ShapeDtypeStruct((N, 2), jnp.int32),
                   jax.ShapeDtypeStruct((2, N), jnp.int32)),
        scratch_shapes=[
            pltpu.VMEM((N, H), jnp.float32),
            pltpu.VMEM((N, H), jnp.float32),
            pltpu.VMEM((N, H), jnp.float32),
            pltpu.VMEM((N, DMAX), jnp.float32),
        ],
    )(x, z2, ew1, eb1, ew2, eb2, pw, pb, sw, tw, dcol, d2, d2)


def _sc_gather(h, idx_flat):
    m = idx_flat.shape[0]
    bpw = m // 32
    nch = bpw // GCH
    mesh = plsc.VectorSubcoreMesh(core_axis_name="c", subcore_axis_name="s")

    @functools.partial(
        pl.kernel, mesh=mesh,
        out_type=jax.ShapeDtypeStruct((m, H), jnp.float32),
        scratch_types=[
            pltpu.VMEM((bpw,), jnp.int32),
            pltpu.VMEM((GCH, H), jnp.float32),
            pltpu.VMEM((GCH, H), jnp.float32),
            pltpu.SemaphoreType.DMA,
            pltpu.SemaphoreType.DMA,
            pltpu.SemaphoreType.DMA,
            pltpu.SemaphoreType.DMA,
        ],
    )
    def kern(h_hbm, i_hbm, o_hbm, idx_v, buf0, buf1, g0, g1, s0, s1):
        wid = jax.lax.axis_index("s") * 2 + jax.lax.axis_index("c")
        base = wid * bpw
        pltpu.sync_copy(i_hbm.at[pl.ds(base, bpw)], idx_v)
        bufs = (buf0, buf1)
        gs = (g0, g1)
        ss = (s0, s1)
        gcp = [None] * nch
        ocp = [None] * nch
        gcp[0] = pltpu.async_copy(h_hbm.at[idx_v.at[pl.ds(0, GCH)]],
                                  bufs[0], gs[0])
        for c in range(nch):
            b = c % 2
            if c + 1 < nch:
                if c >= 1:
                    ocp[c - 1].wait()
                gcp[c + 1] = pltpu.async_copy(
                    h_hbm.at[idx_v.at[pl.ds((c + 1) * GCH, GCH)]],
                    bufs[(c + 1) % 2], gs[(c + 1) % 2])
            gcp[c].wait()
            ocp[c] = pltpu.async_copy(
                bufs[b], o_hbm.at[pl.ds(base + c * GCH, GCH)], ss[b])
        if nch >= 2:
            ocp[nch - 2].wait()
        ocp[nch - 1].wait()

    return kern(h, idx_flat)


def _final_body(g_ref, h_ref, z_ref, iw1_ref, ib1_ref, iw2_ref, ib2_ref,
                nt_ref, vals_ref, idx_ref, src_out, dst_out, attr_out,
                logit_out):
    dn = (((1,), (1,)), ((), ()))
    h = h_ref[...]
    zrow = jnp.broadcast_to(z_ref[...], (N, Z))
    t = nt_ref[...]
    kp = jnp.where(t == 2, 2, jnp.where(t == 1, 1, 0))
    vrow = jax.lax.broadcasted_iota(jnp.int32, (N, 1), 0)
    for j in (0, 1):
        u_emb = g_ref[pl.ds(j * N, N), :]
        feat = jnp.concatenate([u_emb, h, zrow], axis=1)
        a = jnp.maximum(
            jax.lax.dot_general(feat, iw1_ref[...], dn,
                                preferred_element_type=jnp.float32)
            + ib1_ref[...], 0.0)
        logitp = jax.lax.dot_general(a, iw2_ref[...], dn,
                                     preferred_element_type=jnp.float32) \
            + ib2_ref[...]
        logit = logitp[:, 0:1]
        bit = (logit > 0.0).astype(jnp.int32)
        validj = (j < kp) & (vals_ref[:, j:j + 1] > -1e8)
        idxj = idx_ref[:, j:j + 1]
        src_out[:, j:j + 1] = jnp.where(validj, idxj, -1)
        dst_out[:, j:j + 1] = jnp.where(validj, vrow, -1)
        attr_out[:, j:j + 1] = jnp.where(validj, bit, 0)
        logit_out[:, j:j + 1] = logit


def _final_call(g, h, z2, iw1, ib1, iw2p, ib2, nt2, vals, idx):
    i2 = jax.ShapeDtypeStruct((N, 2), jnp.int32)
    f2 = jax.ShapeDtypeStruct((N, 2), jnp.float32)
    return pl.pallas_call(
        _final_body, out_shape=(i2, i2, i2, f2),
    )(g, h, z2, iw1, ib1, iw2p, ib2, nt2, vals, idx)


def kernel(x, z, node_type, node_depth, enc_w1, enc_b1, enc_w2, enc_b2,
           proj_w, proj_b, src_w, tgt_w, inv_w1, inv_b1, inv_w2, inv_b2):
    z2 = z.reshape(1, Z)
    eb1 = enc_b1.reshape(1, H)
    eb2 = enc_b2.reshape(1, H)
    pb = proj_b.reshape(1, H)
    ib1 = inv_b1.reshape(1, H)
    iw2p = jnp.concatenate([inv_w2, jnp.zeros((127, H), jnp.float32)], axis=0)
    ib2 = jnp.broadcast_to(inv_b2.reshape(1, 1), (1, 128))
    d2 = node_depth.reshape(1, N).astype(jnp.int32)
    dcol = node_depth.reshape(N, 1).astype(jnp.int32)
    nt2 = node_type.reshape(N, 1).astype(jnp.int32)

    h, vals, idx, idxf = _fused_call(x, z2, enc_w1, eb1, enc_w2, eb2,
                                     proj_w, pb, src_w, tgt_w, dcol, d2)
    g = _sc_gather(h, idxf.reshape(2 * N))
    edge_src, edge_dst, edge_attr, inv_logit = _final_call(
        g, h, z2, inv_w1, ib1, iw2p, ib2, nt2, vals, idx)
    return edge_src, edge_dst, edge_attr, vals, inv_logit
